# baseline (speedup 1.0000x reference)
import functools

import jax
import jax.numpy as jnp
from jax import lax
from jax.experimental import pallas as pl
from jax.experimental.pallas import tpu as pltpu
from jax.experimental.pallas import tpu_sc as plsc

D = 128
DW = D // 2
CTX = 10
NWALK = 45056
NSUB = 16
WPS = NWALK // NSUB
CHUNK = 32
NCHUNK = WPS // CHUNK
ROWS = CHUNK * CTX
GGRP = 80
EPS = 1e-15
NSCORE = NWALK * 9
OUT_ROWS = 2 * NWALK * 16 // 128


def _sc_body(pos_hbm, neg_hbm, z_hbm, out_hbm,
             idx_all, rows0, rows1, scores0, scores1, sg0, sg1, so0, so1):
    cid = lax.axis_index("c")
    sid = lax.axis_index("s")
    base_walk = cid * NWALK + sid * WPS
    rows = (rows0, rows1)
    scores = (scores0, scores1)
    semg = (sg0, sg1)
    semo = (so0, so1)
    lane = lax.iota(jnp.int32, 16)

    @pl.when(cid == 0)
    def _():
        pltpu.sync_copy(pos_hbm.at[pl.ds(sid * WPS * CTX, WPS * CTX)], idx_all)

    @pl.when(cid == 1)
    def _():
        pltpu.sync_copy(neg_hbm.at[pl.ds(sid * WPS * CTX, WPS * CTX)], idx_all)

    def fetch(g, b):
        for k in range(ROWS // GGRP):
            pltpu.async_copy(
                z_hbm.at[idx_all.at[pl.ds(g * ROWS + k * GGRP, GGRP)]],
                rows[b].at[pl.ds(k * GGRP, GGRP), :],
                semg[b])

    def wait_rows(b):
        pltpu.make_async_copy(z_hbm.at[pl.ds(0, ROWS)], rows[b], semg[b]).wait()

    def wait_out(s):
        pltpu.make_async_copy(scores[s], out_hbm.at[pl.ds(0, 8), :],
                              semo[s]).wait()

    def compute(g, b):
        rb = rows[b % 2]
        sb = scores[b // 2]
        rbase = (b % 2) * 4

        def load_f32(r, c):
            x = rb[r, pl.ds(c * 16, 16)]
            lo = plsc.bitcast(x << 16, jnp.float32)
            hi = plsc.bitcast(x & jnp.int32(-65536), jnp.float32)
            return lo, hi

        def walk_body(w, carry):
            r0 = w * CTX
            h0 = []
            for c in range(DW // 16):
                h0.extend(load_f32(r0, c))
            vec = jnp.zeros((16,), jnp.float32)
            for j in range(1, CTX):
                acc = None
                for c in range(DW // 16):
                    u0, u1 = load_f32(r0 + j, c)
                    t = h0[2 * c] * u0 + h0[2 * c + 1] * u1
                    acc = t if acc is None else acc + t
                for k in (8, 4, 2, 1):
                    acc = acc + acc.at[lane ^ k].get(mode="promise_in_bounds")
                vec = jnp.where(lane == (j - 1), acc, vec)
            sb[rbase + w // 8, pl.ds((w % 8) * 16, 16)] = vec
            return carry

        lax.fori_loop(0, CHUNK, walk_body, 0)
        if b % 2 == 1:
            row0 = pl.multiple_of((base_walk + (g - 1) * CHUNK) * 16 // 128, 8)
            pltpu.async_copy(sb, out_hbm.at[pl.ds(row0, 8), :], semo[b // 2])

    fetch(0, 0)

    def quad_body(q, carry):
        for b in range(4):
            g = q * 4 + b

            @pl.when(g + 1 < NCHUNK)
            def _():
                fetch(g + 1, 1 - b % 2)

            wait_rows(b % 2)

            if b % 2 == 0:
                @pl.when(q >= 1)
                def _():
                    wait_out(b // 2)

            compute(g, b)
        return carry

    lax.fori_loop(0, NCHUNK // 4, quad_body, 0)
    wait_out(0)
    wait_out(1)


_sc_scores = pl.kernel(
    _sc_body,
    out_type=jax.ShapeDtypeStruct((OUT_ROWS, 128), jnp.float32),
    mesh=plsc.VectorSubcoreMesh(core_axis_name="c", subcore_axis_name="s",
                                num_cores=2, num_subcores=16),
    compiler_params=pltpu.CompilerParams(needs_layout_passes=False,
                                         use_tc_tiling_on_sc=False),
    scratch_types=[
        pltpu.VMEM((WPS * CTX,), jnp.int32),
        pltpu.VMEM((ROWS, DW), jnp.int32),
        pltpu.VMEM((ROWS, DW), jnp.int32),
        pltpu.VMEM((8, 128), jnp.float32),
        pltpu.VMEM((8, 128), jnp.float32),
        pltpu.SemaphoreType.DMA,
        pltpu.SemaphoreType.DMA,
        pltpu.SemaphoreType.DMA,
        pltpu.SemaphoreType.DMA,
    ],
)


_TC_BLK = 1024
_TC_GRID = OUT_ROWS // _TC_BLK


def _loss_body(x_ref, o_ref):
    i = pl.program_id(0)
    x = x_ref[...]
    lane = lax.broadcasted_iota(jnp.int32, x.shape, 1)
    row = lax.broadcasted_iota(jnp.int32, x.shape, 0) + i * _TC_BLK
    valid = (lane % 16) < 9
    sig = jax.nn.sigmoid(x)
    arg = jnp.where(row < OUT_ROWS // 2, sig, 1.0 - sig)
    t = -jnp.log(jnp.maximum(arg, EPS))
    t = jnp.where(valid, t, 0.0)
    bs = jnp.sum(t)

    @pl.when(i == 0)
    def _():
        o_ref[0, 0] = 0.0

    acc = o_ref[0, 0] + bs
    o_ref[0, 0] = jnp.where(i == _TC_GRID - 1, acc * (1.0 / NSCORE), acc)


_loss_tc = pl.pallas_call(
    _loss_body,
    grid=(_TC_GRID,),
    in_specs=[pl.BlockSpec((_TC_BLK, 128), lambda i: (i, 0))],
    out_specs=pl.BlockSpec((1, 1), lambda i: (0, 0), memory_space=pltpu.SMEM),
    out_shape=jax.ShapeDtypeStruct((1, 1), jnp.float32),
)


def kernel(z, pos_rw, neg_rw):
    zi = jax.lax.bitcast_convert_type(
        z.astype(jnp.bfloat16).reshape(-1, DW, 2), jnp.int32)
    scores = _sc_scores(pos_rw.reshape(-1), neg_rw.reshape(-1), zi)
    return _loss_tc(scores).reshape(())


# speedup vs baseline: 2.6252x; 2.6252x over previous
import functools
import math

import jax
import jax.numpy as jnp
from jax import lax
from jax.experimental import pallas as pl
from jax.experimental.pallas import tpu as pltpu
from jax.experimental.pallas import tpu_sc as plsc

D = 128
CTX = 10
NWALK = 45056
NSUB = 16
WPS = NWALK // NSUB
CHUNK = 32
NCHUNK = WPS // CHUNK
ROWS = CHUNK * CTX
GGRP = 80
EPS = 1e-15
NSCORE = NWALK * 9
OUT_ROWS = 2 * NWALK * 16 // 128


def _sc_body(pos_hbm, neg_hbm, z_hbm, out_hbm,
             idx_all, rows0, rows1, scores0, scores1, sg0, sg1, so0, so1):
    cid = lax.axis_index("c")
    sid = lax.axis_index("s")
    base_walk = cid * NWALK + sid * WPS
    rows = (rows0, rows1)
    scores = (scores0, scores1)
    semg = (sg0, sg1)
    semo = (so0, so1)
    lane = lax.iota(jnp.int32, 16)

    @pl.when(cid == 0)
    def _():
        pltpu.sync_copy(pos_hbm.at[pl.ds(sid * WPS * CTX, WPS * CTX)], idx_all)

    @pl.when(cid == 1)
    def _():
        pltpu.sync_copy(neg_hbm.at[pl.ds(sid * WPS * CTX, WPS * CTX)], idx_all)

    def fetch(g, b):
        for k in range(ROWS // GGRP):
            pltpu.async_copy(
                z_hbm.at[idx_all.at[pl.ds(g * ROWS + k * GGRP, GGRP)]],
                rows[b].at[pl.ds(k * GGRP, GGRP), :],
                semg[b])

    def wait_rows(b):
        pltpu.make_async_copy(z_hbm.at[pl.ds(0, ROWS)], rows[b], semg[b]).wait()

    def wait_out(s):
        pltpu.make_async_copy(scores[s], out_hbm.at[pl.ds(0, 8), :],
                              semo[s]).wait()

    def compute(g, b):
        rb = rows[b % 2]
        sb = scores[b // 2]
        rbase = (b % 2) * 4

        def walk_body(w, carry):
            r0 = w * CTX
            h0 = [rb[r0, pl.ds(c * 16, 16)] for c in range(D // 16)]
            vec = jnp.zeros((16,), jnp.float32)
            for j in range(1, CTX):
                acc = None
                for c in range(D // 16):
                    t = h0[c] * rb[r0 + j, pl.ds(c * 16, 16)]
                    acc = t if acc is None else acc + t
                for k in (8, 4, 2, 1):
                    acc = acc + acc.at[lane ^ k].get(mode="promise_in_bounds")
                vec = jnp.where(lane == (j - 1), acc, vec)
            sb[rbase + w // 8, pl.ds((w % 8) * 16, 16)] = vec
            return carry

        lax.fori_loop(0, CHUNK, walk_body, 0)
        if b % 2 == 1:
            row0 = pl.multiple_of((base_walk + (g - 1) * CHUNK) * 16 // 128, 8)
            pltpu.async_copy(sb, out_hbm.at[pl.ds(row0, 8), :], semo[b // 2])

    fetch(0, 0)

    def quad_body(q, carry):
        for b in range(4):
            g = q * 4 + b

            @pl.when(g + 1 < NCHUNK)
            def _():
                fetch(g + 1, 1 - b % 2)

            wait_rows(b % 2)

            if b % 2 == 0:
                @pl.when(q >= 1)
                def _():
                    wait_out(b // 2)

            compute(g, b)
        return carry

    lax.fori_loop(0, NCHUNK // 4, quad_body, 0)
    wait_out(0)
    wait_out(1)


_sc_scores = pl.kernel(
    _sc_body,
    out_type=jax.ShapeDtypeStruct((OUT_ROWS, 128), jnp.float32),
    mesh=plsc.VectorSubcoreMesh(core_axis_name="c", subcore_axis_name="s",
                                num_cores=2, num_subcores=16),
    scratch_types=[
        pltpu.VMEM((WPS * CTX,), jnp.int32),
        pltpu.VMEM((ROWS, D), jnp.float32),
        pltpu.VMEM((ROWS, D), jnp.float32),
        pltpu.VMEM((8, 128), jnp.float32),
        pltpu.VMEM((8, 128), jnp.float32),
        pltpu.SemaphoreType.DMA,
        pltpu.SemaphoreType.DMA,
        pltpu.SemaphoreType.DMA,
        pltpu.SemaphoreType.DMA,
    ],
)


_TC_BLK = 1024
_TC_GRID = OUT_ROWS // _TC_BLK


def _loss_body(x_ref, o_ref):
    i = pl.program_id(0)
    x = x_ref[...]
    lane = lax.broadcasted_iota(jnp.int32, x.shape, 1)
    row = lax.broadcasted_iota(jnp.int32, x.shape, 0) + i * _TC_BLK
    valid = (lane % 16) < 9
    sig = jax.nn.sigmoid(x)
    arg = jnp.where(row < OUT_ROWS // 2, sig, 1.0 - sig)
    t = -jnp.log(jnp.maximum(arg, EPS))
    t = jnp.where(valid, t, 0.0)
    bs = jnp.sum(t)

    @pl.when(i == 0)
    def _():
        o_ref[0, 0] = 0.0

    acc = o_ref[0, 0] + bs
    o_ref[0, 0] = jnp.where(i == _TC_GRID - 1, acc * (1.0 / NSCORE), acc)


_loss_tc = pl.pallas_call(
    _loss_body,
    grid=(_TC_GRID,),
    in_specs=[pl.BlockSpec((_TC_BLK, 128), lambda i: (i, 0))],
    out_specs=pl.BlockSpec((1, 1), lambda i: (0, 0), memory_space=pltpu.SMEM),
    out_shape=jax.ShapeDtypeStruct((1, 1), jnp.float32),
)


def kernel(z, pos_rw, neg_rw):
    scores = _sc_scores(pos_rw.reshape(-1), neg_rw.reshape(-1), z)
    return _loss_tc(scores).reshape(())
